# baseline (speedup 1.0000x reference)
import functools

import jax
import jax.numpy as jnp
from jax import lax
from jax.experimental import pallas as pl
from jax.experimental.pallas import tpu as pltpu
from jax.experimental.pallas import tpu_sc as plsc


def _sc_gather_pair(E_user, E_item, user, item):
    B = user.shape[0]
    D = E_user.shape[1]
    info = plsc.get_sparse_core_info()
    NC, NS = info.num_cores, info.num_subcores
    NW = NC * NS
    bpw = B // NW
    CH = 128
    NCH = bpw // CH

    user3 = user.reshape(NW, NCH, CH).astype(jnp.int32)
    item3 = item.reshape(NW, NCH, CH).astype(jnp.int32)

    mesh = plsc.VectorSubcoreMesh(core_axis_name="c", subcore_axis_name="s")

    @functools.partial(
        pl.kernel,
        mesh=mesh,
        out_type=[
            jax.ShapeDtypeStruct((B, D), jnp.float32),
            jax.ShapeDtypeStruct((B, D), jnp.float32),
        ],
        scratch_types=[
            pltpu.VMEM((NCH, CH), jnp.int32),
            pltpu.VMEM((NCH, CH), jnp.int32),
            pltpu.VMEM((bpw, D), jnp.float32),
            pltpu.VMEM((bpw, D), jnp.float32),
            pltpu.SemaphoreType.DMA,
            pltpu.SemaphoreType.DMA,
        ],
        compiler_params=pltpu.CompilerParams(use_tc_tiling_on_sc=False),
    )
    def gather_kernel(eu_hbm, ei_hbm, u_hbm, i_hbm, ou_hbm, oi_hbm,
                      uidx, iidx, urows, irows, su, si):
        wid = lax.axis_index("s") * NC + lax.axis_index("c")
        base = wid * bpw
        pltpu.sync_copy(u_hbm.at[wid], uidx)
        pltpu.sync_copy(i_hbm.at[wid], iidx)
        ucps = [
            pltpu.async_copy(eu_hbm.at[uidx.at[c]],
                             urows.at[pl.ds(c * CH, CH)], su)
            for c in range(NCH)
        ]
        icps = [
            pltpu.async_copy(ei_hbm.at[iidx.at[c]],
                             irows.at[pl.ds(c * CH, CH)], si)
            for c in range(NCH)
        ]
        for cp in ucps:
            cp.wait()
        pltpu.sync_copy(urows, ou_hbm.at[pl.ds(base, bpw)])
        for cp in icps:
            cp.wait()
        pltpu.sync_copy(irows, oi_hbm.at[pl.ds(base, bpw)])

    return gather_kernel(E_user, E_item, user3, item3)


def _tc_dense_assemble(uf, itf, WuT, WiT, bu, bi, emb_u, emb_i):
    B, F = uf.shape
    D = WuT.shape[1]
    bB = 2048

    def body(uf_ref, if_ref, wu_ref, wi_ref, bu_ref, bi_ref,
             eu_ref, ei_ref, o_ref):
        du = jnp.dot(uf_ref[...], wu_ref[...],
                     preferred_element_type=jnp.float32) + bu_ref[...]
        di = jnp.dot(if_ref[...], wi_ref[...],
                     preferred_element_type=jnp.float32) + bi_ref[...]
        o_ref[...] = jnp.concatenate(
            [du, di, eu_ref[...], ei_ref[...]], axis=-1)

    out = pl.pallas_call(
        body,
        grid=(B // bB,),
        in_specs=[
            pl.BlockSpec((bB, F), lambda i: (i, 0)),
            pl.BlockSpec((bB, F), lambda i: (i, 0)),
            pl.BlockSpec((F, D), lambda i: (0, 0)),
            pl.BlockSpec((F, D), lambda i: (0, 0)),
            pl.BlockSpec((1, D), lambda i: (0, 0)),
            pl.BlockSpec((1, D), lambda i: (0, 0)),
            pl.BlockSpec((bB, D), lambda i: (i, 0)),
            pl.BlockSpec((bB, D), lambda i: (i, 0)),
        ],
        out_specs=pl.BlockSpec((bB, 4 * D), lambda i: (i, 0)),
        out_shape=jax.ShapeDtypeStruct((B, 4 * D), jnp.float32),
        compiler_params=pltpu.CompilerParams(
            dimension_semantics=("arbitrary",),
        ),
    )(uf, itf, WuT, WiT, bu, bi, emb_u, emb_i)
    return out


def kernel(users_features, items_features, user, item,
           W_user, b_user, W_item, b_item, E_user, E_item):
    B = users_features.shape[0]
    D = W_user.shape[0]
    emb_u, emb_i = _sc_gather_pair(E_user, E_item, user, item)
    out = _tc_dense_assemble(
        users_features, items_features,
        W_user.T, W_item.T,
        b_user.reshape(1, D), b_item.reshape(1, D),
        emb_u, emb_i,
    )
    return out.reshape(B, 4, D)
